# Initial kernel scaffold; baseline (speedup 1.0000x reference)
import jax
import jax.numpy as jnp
from jax.experimental import pallas as pl


def kernel(x, embeds):
    raise NotImplementedError("write your pallas kernel here")


# speedup vs baseline: 2.0942x; 2.0942x over previous
import jax
import jax.numpy as jnp
from jax.experimental import pallas as pl

NUM_SPACE = 8
CONCEPT_DIM = 32
NUM_CONCEPT = 1024
TOK_TILE = 224
NUM_TOK_TILES = 7


def _qlayer_body(x_ref, e_ref, zq_ref, zk_ref):
    z = x_ref[0]
    E = e_ref[0]
    scores = jax.lax.dot_general(
        E, z, (((1,), (1,)), ((), ())),
        precision=jax.lax.Precision.HIGHEST,
        preferred_element_type=jnp.float32)
    en = jnp.sum(E * E, axis=1)
    d = en[:, None] - 2.0 * scores
    zk = jnp.argmin(d, axis=0).astype(jnp.int32)
    zk_ref[0, 0, :] = zk
    iota = jax.lax.broadcasted_iota(jnp.int32, d.shape, 0)
    onehot = (iota == zk[None, :]).astype(jnp.float32)
    zq_ref[0] = jax.lax.dot_general(
        onehot, E, (((0,), (0,)), ((), ())),
        precision=jax.lax.Precision.HIGHEST,
        preferred_element_type=jnp.float32)


def kernel(x, embeds):
    B, H, W, C = x.shape
    T = B * H * W
    x3 = x.reshape(T, NUM_SPACE, CONCEPT_DIM).transpose(1, 0, 2)
    zq, zk = pl.pallas_call(
        _qlayer_body,
        grid=(NUM_SPACE, NUM_TOK_TILES),
        in_specs=[
            pl.BlockSpec((1, TOK_TILE, CONCEPT_DIM), lambda i, j: (i, j, 0)),
            pl.BlockSpec((1, NUM_CONCEPT, CONCEPT_DIM), lambda i, j: (i, 0, 0)),
        ],
        out_specs=[
            pl.BlockSpec((1, TOK_TILE, CONCEPT_DIM), lambda i, j: (i, j, 0)),
            pl.BlockSpec((1, 1, TOK_TILE),
                         lambda i, j: (i * NUM_TOK_TILES + j, 0, 0)),
        ],
        out_shape=[
            jax.ShapeDtypeStruct((NUM_SPACE, T, CONCEPT_DIM), jnp.float32),
            jax.ShapeDtypeStruct((NUM_SPACE * NUM_TOK_TILES, 1, TOK_TILE),
                                 jnp.int32),
        ],
    )(x3, embeds)
    z_q = zq.transpose(1, 0, 2).reshape(B, H, W, C)
    z_k = zk.reshape(NUM_SPACE, B, H * W)
    return (z_q, x, z_k)


# speedup vs baseline: 4.2359x; 2.0227x over previous
import jax
import jax.numpy as jnp
from jax.experimental import pallas as pl
from jax.experimental.pallas import tpu as pltpu

NUM_SPACE = 8
CONCEPT_DIM = 32
NUM_CONCEPT = 1024
TOK_TILE = 224
NUM_TOK_TILES = 7


def _mm_kt(a, b):
    return jax.lax.dot_general(
        a, b, (((1,), (1,)), ((), ())), preferred_element_type=jnp.float32)


def _mm_tc(a, b):
    return jax.lax.dot_general(
        a, b, (((0,), (0,)), ((), ())), preferred_element_type=jnp.float32)


def _split2(v):
    hi = v.astype(jnp.bfloat16)
    lo = (v - hi.astype(jnp.float32)).astype(jnp.bfloat16)
    return hi, lo


def _qlayer_body(x_ref, e_ref, zq_ref, zk_ref, eh_ref, el_ref, en_ref):
    @pl.when(pl.program_id(0) == 0)
    def _prep():
        for i in range(NUM_SPACE):
            E = e_ref[i]
            eh, el = _split2(E)
            eh_ref[i] = eh
            el_ref[i] = el
            en_ref[i] = jnp.sum(E * E, axis=1, keepdims=True)

    for i in range(NUM_SPACE):
        z = x_ref[0][:, i * CONCEPT_DIM:(i + 1) * CONCEPT_DIM]
        zh, zl = _split2(z)
        eh = eh_ref[i]
        el = el_ref[i]
        scores = ((_mm_kt(eh, zh) + _mm_kt(eh, zl))
                  + (_mm_kt(el, zh) + _mm_kt(el, zl)))
        d = en_ref[i] - 2.0 * scores
        zk = jnp.argmin(d, axis=0).astype(jnp.int32)
        zk_ref[0, i, :] = zk
        iota = jax.lax.broadcasted_iota(jnp.int32, d.shape, 0)
        onehot = (iota == zk[None, :]).astype(jnp.bfloat16)
        zq_ref[0, :, i * CONCEPT_DIM:(i + 1) * CONCEPT_DIM] = (
            _mm_tc(onehot, eh) + _mm_tc(onehot, el))


def kernel(x, embeds):
    B, H, W, C = x.shape
    T = B * H * W
    x3 = x.reshape(NUM_TOK_TILES, TOK_TILE, C)
    zq, zk = pl.pallas_call(
        _qlayer_body,
        grid=(NUM_TOK_TILES,),
        in_specs=[
            pl.BlockSpec((1, TOK_TILE, C), lambda j: (j, 0, 0)),
            pl.BlockSpec((NUM_SPACE, NUM_CONCEPT, CONCEPT_DIM),
                         lambda j: (0, 0, 0)),
        ],
        out_specs=[
            pl.BlockSpec((1, TOK_TILE, C), lambda j: (j, 0, 0)),
            pl.BlockSpec((1, NUM_SPACE, TOK_TILE), lambda j: (j, 0, 0)),
        ],
        out_shape=[
            jax.ShapeDtypeStruct((NUM_TOK_TILES, TOK_TILE, C), jnp.float32),
            jax.ShapeDtypeStruct((NUM_TOK_TILES, NUM_SPACE, TOK_TILE),
                                 jnp.int32),
        ],
        scratch_shapes=[
            pltpu.VMEM((NUM_SPACE, NUM_CONCEPT, CONCEPT_DIM), jnp.bfloat16),
            pltpu.VMEM((NUM_SPACE, NUM_CONCEPT, CONCEPT_DIM), jnp.bfloat16),
            pltpu.VMEM((NUM_SPACE, NUM_CONCEPT, 1), jnp.float32),
        ],
    )(x3, embeds)
    z_q = zq.reshape(B, H, W, C)
    z_k = zk.transpose(1, 0, 2).reshape(NUM_SPACE, B, H * W)
    return (z_q, x, z_k)


# speedup vs baseline: 5.4413x; 1.2846x over previous
import functools

import jax
import jax.numpy as jnp
from jax import lax
from jax.experimental import pallas as pl
from jax.experimental.pallas import tpu as pltpu
from jax.experimental.pallas import tpu_sc as plsc

NUM_SPACE = 8
CONCEPT_DIM = 32
NUM_CONCEPT = 1024
TOK_TILE = 224
NUM_TOK_TILES = 7
NUM_TOKENS = TOK_TILE * NUM_TOK_TILES
NUM_ROWS = NUM_TOKENS * NUM_SPACE


def _mm_kt(a, b):
    return jax.lax.dot_general(
        a, b, (((1,), (1,)), ((), ())), preferred_element_type=jnp.float32)


def _split2(v):
    hi = v.astype(jnp.bfloat16)
    lo = (v - hi.astype(jnp.float32)).astype(jnp.bfloat16)
    return hi, lo


def _argmin_body(x_ref, e_ref, zk_ref, idx_ref, eh_ref, el_ref, en_ref):
    @pl.when(pl.program_id(0) == 0)
    def _prep():
        for i in range(NUM_SPACE):
            E = e_ref[i]
            eh, el = _split2(E)
            eh_ref[i] = eh
            el_ref[i] = el
            en_ref[i] = jnp.sum(E * E, axis=1, keepdims=True)

    for i in range(NUM_SPACE):
        z = x_ref[0][:, i * CONCEPT_DIM:(i + 1) * CONCEPT_DIM]
        zh, zl = _split2(z)
        eh = eh_ref[i]
        el = el_ref[i]
        scores = ((_mm_kt(eh, zh) + _mm_kt(eh, zl))
                  + (_mm_kt(el, zh) + _mm_kt(el, zl)))
        d = en_ref[i] - 2.0 * scores
        zk = jnp.argmin(d, axis=0).astype(jnp.int32)
        zk_ref[0, i, :] = zk
        idx_ref[0, :, i] = zk + (i * NUM_CONCEPT)


GATHER_CHUNKS = 4
CHUNK = 98
PAD_D = 128


def _sc_gather(table_hbm, idx_hbm, out_hbm, idx_v, rows_v, sem):
    info = plsc.get_sparse_core_info()
    wid = lax.axis_index("s") * info.num_cores + lax.axis_index("c")
    pltpu.sync_copy(idx_hbm.at[wid], idx_v)
    copies = [
        pltpu.async_copy(table_hbm.at[idx_v.at[c]], rows_v.at[c], sem)
        for c in range(GATHER_CHUNKS)
    ]
    for cp in copies:
        cp.wait()
    pltpu.sync_copy(rows_v, out_hbm.at[wid])


def kernel(x, embeds):
    B, H, W, C = x.shape
    x3 = x.reshape(NUM_TOK_TILES, TOK_TILE, C)
    zk, idx = pl.pallas_call(
        _argmin_body,
        grid=(NUM_TOK_TILES,),
        in_specs=[
            pl.BlockSpec((1, TOK_TILE, C), lambda j: (j, 0, 0)),
            pl.BlockSpec((NUM_SPACE, NUM_CONCEPT, CONCEPT_DIM),
                         lambda j: (0, 0, 0)),
        ],
        out_specs=[
            pl.BlockSpec((1, NUM_SPACE, TOK_TILE), lambda j: (j, 0, 0)),
            pl.BlockSpec((1, TOK_TILE, NUM_SPACE), lambda j: (j, 0, 0)),
        ],
        out_shape=[
            jax.ShapeDtypeStruct((NUM_TOK_TILES, NUM_SPACE, TOK_TILE),
                                 jnp.int32),
            jax.ShapeDtypeStruct((NUM_TOK_TILES, TOK_TILE, NUM_SPACE),
                                 jnp.int32),
        ],
        scratch_shapes=[
            pltpu.VMEM((NUM_SPACE, NUM_CONCEPT, CONCEPT_DIM), jnp.bfloat16),
            pltpu.VMEM((NUM_SPACE, NUM_CONCEPT, CONCEPT_DIM), jnp.bfloat16),
            pltpu.VMEM((NUM_SPACE, NUM_CONCEPT, 1), jnp.float32),
        ],
    )(x3, embeds)

    info = plsc.get_sparse_core_info()
    nw = info.num_cores * info.num_subcores
    mesh = plsc.VectorSubcoreMesh(core_axis_name="c", subcore_axis_name="s")
    gather = functools.partial(
        pl.kernel, mesh=mesh,
        out_type=jax.ShapeDtypeStruct((nw, GATHER_CHUNKS, CHUNK, PAD_D),
                                      jnp.float32),
        scratch_types=[
            pltpu.VMEM((GATHER_CHUNKS, CHUNK), jnp.int32),
            pltpu.VMEM((GATHER_CHUNKS, CHUNK, PAD_D), jnp.float32),
            pltpu.SemaphoreType.DMA,
        ],
    )(_sc_gather)
    table_pad = jnp.pad(
        embeds.reshape(NUM_SPACE * NUM_CONCEPT, CONCEPT_DIM),
        ((0, 0), (0, PAD_D - CONCEPT_DIM)))
    rows = gather(table_pad, idx.reshape(nw, GATHER_CHUNKS, CHUNK))

    z_q = rows.reshape(NUM_ROWS, PAD_D)[:, :CONCEPT_DIM].reshape(B, H, W, C)
    z_k = zk.transpose(1, 0, 2).reshape(NUM_SPACE, B, H * W)
    return (z_q, x, z_k)


# speedup vs baseline: 6.5630x; 1.2062x over previous
import functools

import jax
import jax.numpy as jnp
from jax import lax
from jax.experimental import pallas as pl
from jax.experimental.pallas import tpu as pltpu
from jax.experimental.pallas import tpu_sc as plsc

NUM_SPACE = 8
CONCEPT_DIM = 32
NUM_CONCEPT = 1024
TOK_TILE = 224
NUM_TOK_TILES = 7
NUM_TOKENS = TOK_TILE * NUM_TOK_TILES
NUM_ROWS = NUM_TOKENS * NUM_SPACE


def _mm_kt(a, b):
    return jax.lax.dot_general(
        a, b, (((1,), (1,)), ((), ())), preferred_element_type=jnp.float32)


def _split2(v):
    hi = v.astype(jnp.bfloat16)
    lo = (v - hi.astype(jnp.float32)).astype(jnp.bfloat16)
    return hi, lo


def _argmin_body(x_ref, e_ref, zk_ref, ea_ref, en_ref):
    @pl.when(pl.program_id(0) == 0)
    def _prep():
        for i in range(NUM_SPACE):
            E = e_ref[i]
            eh, el = _split2(E)
            ea_ref[i] = jnp.concatenate([eh, eh, el, el], axis=1)
            en_ref[i] = jnp.sum(E * E, axis=1, keepdims=True)

    xh, xl = _split2(x_ref[0])
    for i in range(NUM_SPACE):
        sl = slice(i * CONCEPT_DIM, (i + 1) * CONCEPT_DIM)
        zc = jnp.concatenate([xh[:, sl], xl[:, sl]], axis=1)
        za = jnp.concatenate([zc, zc], axis=1)
        scores = _mm_kt(ea_ref[i], za)
        d = en_ref[i] - 2.0 * scores
        zk = jnp.argmin(d, axis=0).astype(jnp.int32)
        zk_ref[0, i, :] = zk


GATHER_CHUNKS = 4
CHUNK = 98
PAD_D = 128


def _sc_gather(table_hbm, idx_hbm, out_hbm, idx_v, rows_v, sem):
    info = plsc.get_sparse_core_info()
    wid = lax.axis_index("s") * info.num_cores + lax.axis_index("c")
    pltpu.sync_copy(idx_hbm.at[wid], idx_v)
    copies = [
        pltpu.async_copy(table_hbm.at[idx_v.at[c]], rows_v.at[c], sem)
        for c in range(GATHER_CHUNKS)
    ]
    for cp in copies:
        cp.wait()
    pltpu.sync_copy(rows_v, out_hbm.at[wid])


def kernel(x, embeds):
    B, H, W, C = x.shape
    x3 = x.reshape(NUM_TOK_TILES, TOK_TILE, C)
    zk = pl.pallas_call(
        _argmin_body,
        grid=(NUM_TOK_TILES,),
        in_specs=[
            pl.BlockSpec((1, TOK_TILE, C), lambda j: (j, 0, 0)),
            pl.BlockSpec((NUM_SPACE, NUM_CONCEPT, CONCEPT_DIM),
                         lambda j: (0, 0, 0)),
        ],
        out_specs=pl.BlockSpec((1, NUM_SPACE, TOK_TILE), lambda j: (j, 0, 0)),
        out_shape=jax.ShapeDtypeStruct(
            (NUM_TOK_TILES, NUM_SPACE, TOK_TILE), jnp.int32),
        scratch_shapes=[
            pltpu.VMEM((NUM_SPACE, NUM_CONCEPT, 4 * CONCEPT_DIM),
                       jnp.bfloat16),
            pltpu.VMEM((NUM_SPACE, NUM_CONCEPT, 1), jnp.float32),
        ],
    )(x3, embeds)
    idx = (zk + (jnp.arange(NUM_SPACE, dtype=jnp.int32)[None, :, None]
                 * NUM_CONCEPT)).transpose(0, 2, 1)

    info = plsc.get_sparse_core_info()
    nw = info.num_cores * info.num_subcores
    mesh = plsc.VectorSubcoreMesh(core_axis_name="c", subcore_axis_name="s")
    gather = functools.partial(
        pl.kernel, mesh=mesh,
        out_type=jax.ShapeDtypeStruct((nw, GATHER_CHUNKS, CHUNK, PAD_D),
                                      jnp.float32),
        scratch_types=[
            pltpu.VMEM((GATHER_CHUNKS, CHUNK), jnp.int32),
            pltpu.VMEM((GATHER_CHUNKS, CHUNK, PAD_D), jnp.float32),
            pltpu.SemaphoreType.DMA,
        ],
    )(_sc_gather)
    table_pad = jnp.pad(
        embeds.reshape(NUM_SPACE * NUM_CONCEPT, CONCEPT_DIM),
        ((0, 0), (0, PAD_D - CONCEPT_DIM)))
    rows = gather(table_pad, idx.reshape(nw, GATHER_CHUNKS, CHUNK))

    z_q = rows.reshape(NUM_ROWS, PAD_D)[:, :CONCEPT_DIM].reshape(B, H, W, C)
    z_k = zk.transpose(1, 0, 2).reshape(NUM_SPACE, B, H * W)
    return (z_q, x, z_k)


# speedup vs baseline: 6.5654x; 1.0004x over previous
import functools

import jax
import jax.numpy as jnp
from jax import lax
from jax.experimental import pallas as pl
from jax.experimental.pallas import tpu as pltpu
from jax.experimental.pallas import tpu_sc as plsc

NUM_SPACE = 8
CONCEPT_DIM = 32
NUM_CONCEPT = 1024
TOK_TILE = 224
NUM_TOK_TILES = 7
NUM_TOKENS = TOK_TILE * NUM_TOK_TILES
NUM_ROWS = NUM_TOKENS * NUM_SPACE


def _mm_kt(a, b):
    return jax.lax.dot_general(
        a, b, (((1,), (1,)), ((), ())), preferred_element_type=jnp.float32)


def _split2(v):
    hi = v.astype(jnp.bfloat16)
    lo = (v - hi.astype(jnp.float32)).astype(jnp.bfloat16)
    return hi, lo


def _argmin_rows(d):
    val = d
    idx = jax.lax.broadcasted_iota(jnp.int32, d.shape, 0)
    r = d.shape[0]
    while r > 1:
        h = r // 2
        a, b = val[:h], val[h:]
        ia, ib = idx[:h], idx[h:]
        m = b < a
        val = jnp.where(m, b, a)
        idx = jnp.where(m, ib, ia)
        r = h
    return idx[0]


def _argmin_body(x_ref, e_ref, zk_ref, ea_ref, en_ref):
    @pl.when(pl.program_id(0) == 0)
    def _prep():
        for i in range(NUM_SPACE):
            E = e_ref[i]
            eh, el = _split2(E)
            ea_ref[i] = jnp.concatenate([eh, eh, el, el], axis=1)
            en_ref[i] = jnp.sum(E * E, axis=1, keepdims=True)

    xh, xl = _split2(x_ref[0])
    for i in range(NUM_SPACE):
        sl = slice(i * CONCEPT_DIM, (i + 1) * CONCEPT_DIM)
        zc = jnp.concatenate([xh[:, sl], xl[:, sl]], axis=1)
        za = jnp.concatenate([zc, zc], axis=1)
        scores = _mm_kt(ea_ref[i], za)
        d = en_ref[i] - 2.0 * scores
        zk = jnp.argmin(d, axis=0).astype(jnp.int32)
        zk_ref[0, i, :] = zk


GATHER_CHUNKS = 4
CHUNK = 98
PAD_D = 128


def _sc_gather(table_hbm, idx_hbm, out_hbm, idx_v, rows_v, sem):
    info = plsc.get_sparse_core_info()
    wid = lax.axis_index("s") * info.num_cores + lax.axis_index("c")
    pltpu.sync_copy(idx_hbm.at[wid], idx_v)
    copies = [
        pltpu.async_copy(table_hbm.at[idx_v.at[c]], rows_v.at[c], sem)
        for c in range(GATHER_CHUNKS)
    ]
    for cp in copies:
        cp.wait()
    pltpu.sync_copy(rows_v, out_hbm.at[wid])


def kernel(x, embeds):
    B, H, W, C = x.shape
    x3 = x.reshape(NUM_TOK_TILES, TOK_TILE, C)
    zk = pl.pallas_call(
        _argmin_body,
        grid=(NUM_TOK_TILES,),
        in_specs=[
            pl.BlockSpec((1, TOK_TILE, C), lambda j: (j, 0, 0)),
            pl.BlockSpec((NUM_SPACE, NUM_CONCEPT, CONCEPT_DIM),
                         lambda j: (0, 0, 0)),
        ],
        out_specs=pl.BlockSpec((1, NUM_SPACE, TOK_TILE), lambda j: (j, 0, 0)),
        out_shape=jax.ShapeDtypeStruct(
            (NUM_TOK_TILES, NUM_SPACE, TOK_TILE), jnp.int32),
        scratch_shapes=[
            pltpu.VMEM((NUM_SPACE, NUM_CONCEPT, 4 * CONCEPT_DIM),
                       jnp.bfloat16),
            pltpu.VMEM((NUM_SPACE, NUM_CONCEPT, 1), jnp.float32),
        ],
    )(x3, embeds)
    idx = (zk + (jnp.arange(NUM_SPACE, dtype=jnp.int32)[None, :, None]
                 * NUM_CONCEPT)).transpose(0, 2, 1)

    info = plsc.get_sparse_core_info()
    nw = info.num_cores * info.num_subcores
    mesh = plsc.VectorSubcoreMesh(core_axis_name="c", subcore_axis_name="s")
    gather = functools.partial(
        pl.kernel, mesh=mesh,
        out_type=jax.ShapeDtypeStruct((nw, GATHER_CHUNKS, CHUNK, PAD_D),
                                      jnp.float32),
        scratch_types=[
            pltpu.VMEM((GATHER_CHUNKS, CHUNK), jnp.int32),
            pltpu.VMEM((GATHER_CHUNKS, CHUNK, PAD_D), jnp.float32),
            pltpu.SemaphoreType.DMA,
        ],
    )(_sc_gather)
    table_pad = jnp.pad(
        embeds.reshape(NUM_SPACE * NUM_CONCEPT, CONCEPT_DIM),
        ((0, 0), (0, PAD_D - CONCEPT_DIM)))
    rows = gather(table_pad, idx.reshape(nw, GATHER_CHUNKS, CHUNK))

    z_q = rows.reshape(NUM_ROWS, PAD_D)[:, :CONCEPT_DIM].reshape(B, H, W, C)
    z_k = zk.transpose(1, 0, 2).reshape(NUM_SPACE, B, H * W)
    return (z_q, x, z_k)
